# baseline (speedup 1.0000x reference)
import functools

import jax
import jax.numpy as jnp
from jax import lax
from jax.experimental import pallas as pl
from jax.experimental.pallas import tpu as pltpu
from jax.experimental.pallas import tpu_sc as plsc

VOCAB = 100000
WORD_DIM = 128
CHAR_VOCAB = 262
CHAR_DIM = 16
NUM_FILTERS = 128
KERNEL = 5
W = 16


def _make_sc_gather(B, D, nw):
    b_per_w = B // nw
    CH = 80
    n_ch = b_per_w // CH
    mesh = plsc.VectorSubcoreMesh(core_axis_name="c", subcore_axis_name="s")

    @functools.partial(
        pl.kernel,
        mesh=mesh,
        out_type=jax.ShapeDtypeStruct((B, D), jnp.float32),
        scratch_types=[
            pltpu.VMEM((n_ch, CH), jnp.int32),
            pltpu.VMEM((CH, D), jnp.float32),
            pltpu.VMEM((CH, D), jnp.float32),
            pltpu.SemaphoreType.DMA,
            pltpu.SemaphoreType.DMA,
        ],
    )
    def sc_gather(table_hbm, idx_hbm, out_hbm, idx_v, buf0, buf1, sem0, sem1):
        wid = lax.axis_index("s") * 2 + lax.axis_index("c")
        base = wid * b_per_w
        pltpu.sync_copy(idx_hbm.at[wid], idx_v)
        bufs = (buf0, buf1)
        sems = (sem0, sem1)
        copies = [None] * n_ch
        for j in range(n_ch):
            copies[j] = pltpu.async_copy(
                table_hbm.at[idx_v.at[j]], bufs[j % 2], sems[j % 2])
            if j >= 1:
                copies[j - 1].wait()
                pltpu.sync_copy(bufs[(j - 1) % 2],
                                out_hbm.at[pl.ds(base + (j - 1) * CH, CH)])
        copies[n_ch - 1].wait()
        pltpu.sync_copy(bufs[(n_ch - 1) % 2],
                        out_hbm.at[pl.ds(base + (n_ch - 1) * CH, CH)])

    return sc_gather


def _tc_body(w_ref, cT_ref, tab_ref, w80_ref, b_ref, out_ref):
    T = out_ref.shape[0]
    N = T * W
    out_ref[:, :WORD_DIM] = w_ref[...]

    G = T // 128
    c = cT_ref[...]
    idx_row = jnp.concatenate(
        [c[W * g + p: W * g + p + 1, :]
         for p in range(W) for g in range(G)], axis=1)
    tabT = lax.transpose(tab_ref[...], (1, 0)).astype(jnp.bfloat16)
    wcT = jnp.concatenate(
        [w80_ref[...], b_ref[...]], axis=0).astype(jnp.bfloat16)
    iota_s = lax.broadcasted_iota(jnp.int16, (CHAR_VOCAB, N), 0)
    one = jnp.ones((), jnp.bfloat16)
    zero = jnp.zeros((), jnp.bfloat16)
    ohT = jnp.where(idx_row == iota_s, one, zero)
    ceT = jnp.dot(tabT, ohT,
                  preferred_element_type=jnp.float32
                  ).astype(jnp.bfloat16)

    z2 = jnp.zeros((CHAR_DIM, 2 * T), jnp.bfloat16)
    rows = []
    for k in range(KERNEL):
        d = (k - 2) * T
        if d < 0:
            rows.append(jnp.concatenate([z2[:, :-d], ceT[:, :N + d]], axis=1))
        elif d == 0:
            rows.append(ceT)
        else:
            rows.append(jnp.concatenate([ceT[:, d:], z2[:, :d]], axis=1))
    rows.append(jnp.ones((1, N), jnp.bfloat16))
    xT = jnp.concatenate(rows, axis=0)
    yT = lax.dot_general(wcT, xT, (((0,), (0,)), ((), ())),
                         preferred_element_type=jnp.float32)

    acc = yT[:, :T]
    for p in range(1, W):
        acc = jnp.maximum(acc, yT[:, p * T:(p + 1) * T])
    m = jnp.maximum(acc, 0.0)
    out_ref[:, WORD_DIM:] = lax.transpose(m, (1, 0))


def _make_tc(B, T):
    grid = (B // T,)
    return pl.pallas_call(
        _tc_body,
        grid=grid,
        in_specs=[
            pl.BlockSpec((T, WORD_DIM), lambda i: (i, 0)),
            pl.BlockSpec((T // 8, 128), lambda i: (i, 0)),
            pl.BlockSpec((CHAR_VOCAB, CHAR_DIM), lambda i: (0, 0)),
            pl.BlockSpec((KERNEL * CHAR_DIM, NUM_FILTERS), lambda i: (0, 0)),
            pl.BlockSpec((1, NUM_FILTERS), lambda i: (0, 0)),
        ],
        out_specs=pl.BlockSpec((T, WORD_DIM + NUM_FILTERS), lambda i: (i, 0)),
        out_shape=jax.ShapeDtypeStruct((B, WORD_DIM + NUM_FILTERS),
                                       jnp.float32),
    )


@jax.jit
def _run(words, chars, word_table, char_table, conv_w, conv_b):
    Bt, L = words.shape
    B = Bt * L
    nw = 32
    idx = words.astype(jnp.int32).reshape(nw, (B // nw) // 80, 80)
    sc_gather = _make_sc_gather(B, WORD_DIM, nw)
    w_filled = sc_gather(word_table, idx)

    T = 1024
    chars_f = (chars.reshape(B // 128, 128, W).astype(jnp.int16)
               .transpose(0, 2, 1).reshape(B // 8, 128))
    w80 = conv_w.reshape(KERNEL * CHAR_DIM, NUM_FILTERS)
    b2 = conv_b.reshape(1, NUM_FILTERS)
    out = _make_tc(B, T)(w_filled, chars_f, char_table, w80, b2)
    return out.reshape(Bt, L, WORD_DIM + NUM_FILTERS)


def kernel(words, chars, word_table, char_table, conv_w, conv_b):
    return _run(words, chars, word_table, char_table, conv_w, conv_b)


# speedup vs baseline: 1.0293x; 1.0293x over previous
import functools

import jax
import jax.numpy as jnp
from jax import lax
from jax.experimental import pallas as pl
from jax.experimental.pallas import tpu as pltpu
from jax.experimental.pallas import tpu_sc as plsc

VOCAB = 100000
WORD_DIM = 128
CHAR_VOCAB = 262
CHAR_DIM = 16
NUM_FILTERS = 128
KERNEL = 5
W = 16


def _make_sc_gather(B, D, nw):
    b_per_w = B // nw
    CH = 80
    n_ch = b_per_w // CH
    mesh = plsc.VectorSubcoreMesh(core_axis_name="c", subcore_axis_name="s")

    @functools.partial(
        pl.kernel,
        mesh=mesh,
        out_type=jax.ShapeDtypeStruct((B, 2 * D), jnp.float32),
        scratch_types=[
            pltpu.VMEM((n_ch, CH), jnp.int32),
            pltpu.VMEM((CH, D), jnp.float32),
            pltpu.VMEM((CH, D), jnp.float32),
            pltpu.SemaphoreType.DMA,
            pltpu.SemaphoreType.DMA,
        ],
    )
    def sc_gather(table_hbm, idx_hbm, out_hbm, idx_v, buf0, buf1, sem0, sem1):
        wid = lax.axis_index("s") * 2 + lax.axis_index("c")
        base = wid * b_per_w
        pltpu.sync_copy(idx_hbm.at[wid], idx_v)
        bufs = (buf0, buf1)
        sems = (sem0, sem1)
        copies = [None] * n_ch
        for j in range(n_ch):
            copies[j] = pltpu.async_copy(
                table_hbm.at[idx_v.at[j]], bufs[j % 2], sems[j % 2])
            if j >= 1:
                copies[j - 1].wait()
                pltpu.sync_copy(
                    bufs[(j - 1) % 2],
                    out_hbm.at[pl.ds(base + (j - 1) * CH, CH), pl.ds(0, D)])
        copies[n_ch - 1].wait()
        pltpu.sync_copy(
            bufs[(n_ch - 1) % 2],
            out_hbm.at[pl.ds(base + (n_ch - 1) * CH, CH), pl.ds(0, D)])

    return sc_gather


def _tc_body(w_ref, cT_ref, tab_ref, w80_ref, b_ref, out_ref):
    del w_ref
    T = out_ref.shape[0]
    N = T * W

    G = T // 128
    c = cT_ref[...]
    idx_row = jnp.concatenate(
        [c[W * g + p: W * g + p + 1, :]
         for p in range(W) for g in range(G)], axis=1)
    tabT = lax.transpose(tab_ref[...], (1, 0)).astype(jnp.bfloat16)
    wcT = jnp.concatenate(
        [w80_ref[...], b_ref[...]], axis=0).astype(jnp.bfloat16)
    iota_s = lax.broadcasted_iota(jnp.int16, (CHAR_VOCAB, N), 0)
    one = jnp.ones((), jnp.bfloat16)
    zero = jnp.zeros((), jnp.bfloat16)
    ohT = jnp.where(idx_row == iota_s, one, zero)
    ceT = jnp.dot(tabT, ohT,
                  preferred_element_type=jnp.float32
                  ).astype(jnp.bfloat16)

    z2 = jnp.zeros((CHAR_DIM, 2 * T), jnp.bfloat16)
    rows = []
    for k in range(KERNEL):
        d = (k - 2) * T
        if d < 0:
            rows.append(jnp.concatenate([z2[:, :-d], ceT[:, :N + d]], axis=1))
        elif d == 0:
            rows.append(ceT)
        else:
            rows.append(jnp.concatenate([ceT[:, d:], z2[:, :d]], axis=1))
    rows.append(jnp.ones((1, N), jnp.bfloat16))
    xT = jnp.concatenate(rows, axis=0)
    yT = lax.dot_general(wcT, xT, (((0,), (0,)), ((), ())),
                         preferred_element_type=jnp.float32)

    acc = yT[:, :T]
    for p in range(1, W):
        acc = jnp.maximum(acc, yT[:, p * T:(p + 1) * T])
    m = jnp.maximum(acc, 0.0)
    out_ref[...] = lax.transpose(m, (1, 0))


def _make_tc(B, T):
    grid = (B // T,)
    return pl.pallas_call(
        _tc_body,
        grid=grid,
        in_specs=[
            pl.BlockSpec(memory_space=pltpu.MemorySpace.HBM),
            pl.BlockSpec((T // 8, 128), lambda i: (i, 0)),
            pl.BlockSpec((CHAR_VOCAB, CHAR_DIM), lambda i: (0, 0)),
            pl.BlockSpec((KERNEL * CHAR_DIM, NUM_FILTERS), lambda i: (0, 0)),
            pl.BlockSpec((1, NUM_FILTERS), lambda i: (0, 0)),
        ],
        out_specs=pl.BlockSpec((T, NUM_FILTERS), lambda i: (i, 1)),
        out_shape=jax.ShapeDtypeStruct((B, WORD_DIM + NUM_FILTERS),
                                       jnp.float32),
        input_output_aliases={0: 0},
    )


@jax.jit
def _run(words, chars, word_table, char_table, conv_w, conv_b):
    Bt, L = words.shape
    B = Bt * L
    nw = 32
    idx = words.astype(jnp.int32).reshape(nw, (B // nw) // 80, 80)
    sc_gather = _make_sc_gather(B, WORD_DIM, nw)
    w_filled = sc_gather(word_table, idx)

    T = 2048
    chars_f = (chars.reshape(B // 128, 128, W).astype(jnp.int16)
               .transpose(0, 2, 1).reshape(B // 8, 128))
    w80 = conv_w.reshape(KERNEL * CHAR_DIM, NUM_FILTERS)
    b2 = conv_b.reshape(1, NUM_FILTERS)
    out = _make_tc(B, T)(w_filled, chars_f, char_table, w80, b2)
    return out.reshape(Bt, L, WORD_DIM + NUM_FILTERS)


def kernel(words, chars, word_table, char_table, conv_w, conv_b):
    return _run(words, chars, word_table, char_table, conv_w, conv_b)


# speedup vs baseline: 1.0297x; 1.0004x over previous
import functools

import jax
import jax.numpy as jnp
from jax import lax
from jax.experimental import pallas as pl
from jax.experimental.pallas import tpu as pltpu
from jax.experimental.pallas import tpu_sc as plsc

VOCAB = 100000
WORD_DIM = 128
CHAR_VOCAB = 262
CHAR_DIM = 16
NUM_FILTERS = 128
KERNEL = 5
W = 16


def _make_sc_gather(B, D, nw):
    b_per_w = B // nw
    CH = 80
    n_ch = b_per_w // CH
    mesh = plsc.VectorSubcoreMesh(core_axis_name="c", subcore_axis_name="s")

    @functools.partial(
        pl.kernel,
        mesh=mesh,
        out_type=jax.ShapeDtypeStruct((B, 2 * D), jnp.float32),
        scratch_types=[
            pltpu.VMEM((n_ch, CH), jnp.int32),
            pltpu.VMEM((CH, D), jnp.float32),
            pltpu.VMEM((CH, D), jnp.float32),
            pltpu.SemaphoreType.DMA,
            pltpu.SemaphoreType.DMA,
        ],
    )
    def sc_gather(table_hbm, idx_hbm, out_hbm, idx_v, buf0, buf1, sem0, sem1):
        wid = lax.axis_index("s") * 2 + lax.axis_index("c")
        base = wid * b_per_w
        pltpu.sync_copy(idx_hbm.at[wid], idx_v)
        bufs = (buf0, buf1)
        sems = (sem0, sem1)
        copies = [None] * n_ch
        for j in range(n_ch):
            copies[j] = pltpu.async_copy(
                table_hbm.at[idx_v.at[j]], bufs[j % 2], sems[j % 2])
            if j >= 1:
                copies[j - 1].wait()
                pltpu.sync_copy(
                    bufs[(j - 1) % 2],
                    out_hbm.at[pl.ds(base + (j - 1) * CH, CH), pl.ds(0, D)])
        copies[n_ch - 1].wait()
        pltpu.sync_copy(
            bufs[(n_ch - 1) % 2],
            out_hbm.at[pl.ds(base + (n_ch - 1) * CH, CH), pl.ds(0, D)])

    return sc_gather


def _tc_body(w_ref, cT_ref, tab_ref, w80_ref, b_ref, out_ref):
    del w_ref
    T = out_ref.shape[0]
    N = T * W

    G = T // 128
    c = cT_ref[...]
    idx_row = jnp.concatenate(
        [c[W * g + p: W * g + p + 1, :]
         for p in range(W) for g in range(G)], axis=1)
    tabT = lax.transpose(tab_ref[...], (1, 0)).astype(jnp.bfloat16)
    wcT = jnp.concatenate(
        [w80_ref[...], b_ref[...]], axis=0).astype(jnp.bfloat16)
    iota_s = lax.broadcasted_iota(jnp.int16, (CHAR_VOCAB, N), 0)
    one = jnp.ones((), jnp.bfloat16)
    zero = jnp.zeros((), jnp.bfloat16)
    ohT = jnp.where(idx_row == iota_s, one, zero)
    ceT = jnp.dot(tabT, ohT,
                  preferred_element_type=jnp.float32
                  ).astype(jnp.bfloat16)

    z2 = jnp.zeros((CHAR_DIM, 2 * T), jnp.bfloat16)
    rows = []
    for k in range(KERNEL):
        d = (k - 2) * T
        if d < 0:
            rows.append(jnp.concatenate([z2[:, :-d], ceT[:, :N + d]], axis=1))
        elif d == 0:
            rows.append(ceT)
        else:
            rows.append(jnp.concatenate([ceT[:, d:], z2[:, :d]], axis=1))
    rows.append(jnp.ones((1, N), jnp.bfloat16))
    xT = jnp.concatenate(rows, axis=0)
    yT = lax.dot_general(wcT, xT, (((0,), (0,)), ((), ())),
                         preferred_element_type=jnp.float32)

    acc = yT[:, :T]
    for p in range(1, W):
        acc = jnp.maximum(acc, yT[:, p * T:(p + 1) * T])
    m = jnp.maximum(acc, 0.0)
    out_ref[...] = lax.transpose(m, (1, 0))


def _make_tc(B, T):
    grid = (B // T,)
    return pl.pallas_call(
        _tc_body,
        grid=grid,
        in_specs=[
            pl.BlockSpec(memory_space=pltpu.MemorySpace.HBM),
            pl.BlockSpec((T // 8, 128), lambda i: (i, 0)),
            pl.BlockSpec((CHAR_VOCAB, CHAR_DIM), lambda i: (0, 0)),
            pl.BlockSpec((KERNEL * CHAR_DIM, NUM_FILTERS), lambda i: (0, 0)),
            pl.BlockSpec((1, NUM_FILTERS), lambda i: (0, 0)),
        ],
        out_specs=pl.BlockSpec((T, NUM_FILTERS), lambda i: (i, 1)),
        out_shape=jax.ShapeDtypeStruct((B, WORD_DIM + NUM_FILTERS),
                                       jnp.float32),
        input_output_aliases={0: 0},
    )


@jax.jit
def _run(words, chars, word_table, char_table, conv_w, conv_b):
    Bt, L = words.shape
    B = Bt * L
    nw = 32
    idx = words.astype(jnp.int32).reshape(nw, (B // nw) // 80, 80)
    sc_gather = _make_sc_gather(B, WORD_DIM, nw)
    w_filled = sc_gather(word_table, idx)

    T = 2560
    chars_f = (chars.reshape(B // 128, 128, W).astype(jnp.int16)
               .transpose(0, 2, 1).reshape(B // 8, 128))
    w80 = conv_w.reshape(KERNEL * CHAR_DIM, NUM_FILTERS)
    b2 = conv_b.reshape(1, NUM_FILTERS)
    out = _make_tc(B, T)(w_filled, chars_f, char_table, w80, b2)
    return out.reshape(Bt, L, WORD_DIM + NUM_FILTERS)


def kernel(words, chars, word_table, char_table, conv_w, conv_b):
    return _run(words, chars, word_table, char_table, conv_w, conv_b)
